# speedup vs baseline: 7.1440x; 7.1440x over previous
import functools

import jax
import jax.numpy as jnp
from jax import lax
from jax.experimental import pallas as pl
from jax.experimental.pallas import tpu as pltpu
from jax.experimental.pallas import tpu_sc as plsc

VOCAB = 1000
EMBD = 150
HOPS = 3
BS = 1024
STORY = 50
SENT = 20
QLEN = 20

DPAD = 160
NT = HOPS + 1
DCAT = NT * DPAD

NC = 2
NS = 16
NW = NC * NS

S_STORY = BS * STORY
S_ALL = S_STORY + BS
PER_W = S_ALL // NW
B_SEG = 4
N_IT = PER_W // B_SEG
NIDX = B_SEG * SENT

_mesh = plsc.VectorSubcoreMesh(core_axis_name="c", subcore_axis_name="s")


@functools.partial(
    pl.kernel,
    out_type=[jax.ShapeDtypeStruct((S_ALL, DPAD), jnp.float32)
              for _ in range(NT)],
    mesh=_mesh,
    scratch_types=[
        pltpu.VMEM((NIDX,), jnp.int32),
        pltpu.VMEM((NIDX, DCAT), jnp.float32),
        pltpu.VMEM((NT, B_SEG, DPAD), jnp.float32),
        pltpu.SemaphoreType.DMA,
    ],
)
def _sc_embed(idx_hbm, tab_hbm, o0, o1, o2, o3, idx_v, rows_v, acc_v, sem):
    outs = (o0, o1, o2, o3)
    wid = lax.axis_index("s") * NC + lax.axis_index("c")
    seg_base = wid * PER_W

    def step(it, carry):
        seg0 = seg_base + it * B_SEG
        pltpu.sync_copy(idx_hbm.at[pl.ds(seg0 * SENT, NIDX)], idx_v)
        pltpu.async_copy(tab_hbm.at[idx_v], rows_v, sem).wait()
        for s in range(B_SEG):
            for j in range(NT):
                def chunk(d, _, s=s, j=j):
                    col = j * DPAD + d * 16
                    a = rows_v[s * SENT, pl.ds(col, 16)]
                    b = rows_v[s * SENT + 1, pl.ds(col, 16)]
                    for t in range(2, SENT, 2):
                        a = a + rows_v[s * SENT + t, pl.ds(col, 16)]
                        b = b + rows_v[s * SENT + t + 1, pl.ds(col, 16)]
                    acc_v[j, s, pl.ds(d * 16, 16)] = a + b
                    return 0
                lax.fori_loop(0, DPAD // 16, chunk, 0)
        for j in range(NT):
            pltpu.sync_copy(acc_v.at[j], outs[j].at[pl.ds(seg0, B_SEG)])
        return carry

    lax.fori_loop(0, N_IT, step, 0)


BB = 64


def _tc_dense(u0_ref, e0, e1, e2, e3, a3t_ref, out_ref):
    es = (e0, e1, e2, e3)
    u = u0_ref[...]
    for k in range(HOPS):
        m = es[k][...]
        c = es[k + 1][...]
        p = jnp.sum(m * u[:, None, :], axis=2)
        p = p - jnp.max(p, axis=1, keepdims=True)
        p = jnp.exp(p)
        p = p / jnp.sum(p, axis=1, keepdims=True)
        u = u + jnp.sum(p[:, :, None] * c, axis=1)
    logits = jnp.dot(u, a3t_ref[...], preferred_element_type=jnp.float32)
    logits = logits - jnp.max(logits, axis=1, keepdims=True)
    z = jnp.exp(logits)
    out_ref[...] = z / jnp.sum(z, axis=1, keepdims=True)


def kernel(x, query, A0, A1, A2, A3):
    tabs = [jnp.pad(t, ((0, 0), (0, DPAD - EMBD))) for t in (A0, A1, A2, A3)]
    tab_cat = jnp.concatenate(tabs, axis=1)
    idx_all = jnp.concatenate(
        [x.reshape(S_STORY, SENT), query], axis=0).reshape(-1)
    a3t = tabs[3].T

    e0, e1, e2, e3 = _sc_embed(idx_all, tab_cat)
    u0 = e0[S_STORY:]
    est = [e[:S_STORY].reshape(BS, STORY, DPAD) for e in (e0, e1, e2, e3)]

    out = pl.pallas_call(
        _tc_dense,
        grid=(BS // BB,),
        in_specs=[
            pl.BlockSpec((BB, DPAD), lambda g: (g, 0)),
            *[pl.BlockSpec((BB, STORY, DPAD), lambda g: (g, 0, 0))
              for _ in range(NT)],
            pl.BlockSpec((DPAD, VOCAB), lambda g: (0, 0)),
        ],
        out_specs=pl.BlockSpec((BB, VOCAB), lambda g: (g, 0)),
        out_shape=jax.ShapeDtypeStruct((BS, VOCAB), jnp.float32),
    )(u0, *est, a3t)
    return out


# speedup vs baseline: 10.6638x; 1.4927x over previous
import functools

import jax
import jax.numpy as jnp
from jax import lax
from jax.experimental import pallas as pl
from jax.experimental.pallas import tpu as pltpu
from jax.experimental.pallas import tpu_sc as plsc

VOCAB = 1000
EMBD = 150
HOPS = 3
BS = 1024
STORY = 50
SENT = 20
QLEN = 20

DPAD = 160
NT = HOPS + 1
DCAT = NT * DPAD

NC = 2
NS = 16
NW = NC * NS

S_STORY = BS * STORY
S_ALL = S_STORY + BS
PER_W = S_ALL // NW
B_SEG = 2
N_IT = PER_W // B_SEG
NIDX = B_SEG * SENT

_mesh = plsc.VectorSubcoreMesh(core_axis_name="c", subcore_axis_name="s")


@functools.partial(
    pl.kernel,
    out_type=[jax.ShapeDtypeStruct((S_ALL, DPAD), jnp.float32)
              for _ in range(NT)],
    mesh=_mesh,
    scratch_types=[
        pltpu.VMEM((PER_W * SENT,), jnp.int32),
        pltpu.VMEM((NIDX, DCAT), jnp.float32),
        pltpu.VMEM((NIDX, DCAT), jnp.float32),
        pltpu.VMEM((NT, B_SEG, DPAD), jnp.float32),
        pltpu.SemaphoreType.DMA,
        pltpu.SemaphoreType.DMA,
    ],
)
def _sc_embed(idx_hbm, tab_hbm, o0, o1, o2, o3, idx_v, rows0, rows1, acc_v,
              sem0, sem1):
    outs = (o0, o1, o2, o3)
    wid = lax.axis_index("s") * NC + lax.axis_index("c")
    seg_base = wid * PER_W

    pltpu.sync_copy(idx_hbm.at[pl.ds(seg_base * SENT, PER_W * SENT)], idx_v)

    def gather(it, rows, sem):
        pltpu.async_copy(
            tab_hbm.at[idx_v.at[pl.ds(it * NIDX, NIDX)]], rows, sem)

    def reduce_store(rows_v, it):
        for s in range(B_SEG):
            for j in range(NT):
                def chunk(d, _, s=s, j=j):
                    col = j * DPAD + d * 16
                    a = rows_v[s * SENT, pl.ds(col, 16)]
                    b = rows_v[s * SENT + 1, pl.ds(col, 16)]
                    for t in range(2, SENT, 2):
                        a = a + rows_v[s * SENT + t, pl.ds(col, 16)]
                        b = b + rows_v[s * SENT + t + 1, pl.ds(col, 16)]
                    acc_v[j, s, pl.ds(d * 16, 16)] = a + b
                    return 0
                lax.fori_loop(0, DPAD // 16, chunk, 0)
        seg0 = seg_base + it * B_SEG
        for j in range(NT):
            pltpu.sync_copy(acc_v.at[j], outs[j].at[pl.ds(seg0, B_SEG)])

    gather(0, rows0, sem0)

    def step(g, carry):
        it0 = 2 * g
        pltpu.make_async_copy(tab_hbm.at[idx_v.at[pl.ds(0, NIDX)]],
                              rows0, sem0).wait()
        gather(it0 + 1, rows1, sem1)
        reduce_store(rows0, it0)
        pltpu.make_async_copy(tab_hbm.at[idx_v.at[pl.ds(0, NIDX)]],
                              rows1, sem1).wait()

        @pl.when(g < N_IT // 2 - 1)
        def _():
            gather(it0 + 2, rows0, sem0)

        reduce_store(rows1, it0 + 1)
        return carry

    lax.fori_loop(0, N_IT // 2, step, 0)


BB = 64


def _tc_dense(u0_ref, e0, e1, e2, e3, a3t_ref, out_ref):
    es = (e0, e1, e2, e3)
    u = u0_ref[...]
    for k in range(HOPS):
        m = es[k][...]
        c = es[k + 1][...]
        p = jnp.sum(m * u[:, None, :], axis=2)
        p = p - jnp.max(p, axis=1, keepdims=True)
        p = jnp.exp(p)
        p = p / jnp.sum(p, axis=1, keepdims=True)
        u = u + jnp.sum(p[:, :, None] * c, axis=1)
    logits = jnp.dot(u, a3t_ref[...], preferred_element_type=jnp.float32)
    logits = logits - jnp.max(logits, axis=1, keepdims=True)
    z = jnp.exp(logits)
    out_ref[...] = z / jnp.sum(z, axis=1, keepdims=True)


def kernel(x, query, A0, A1, A2, A3):
    tabs = [jnp.pad(t, ((0, 0), (0, DPAD - EMBD))) for t in (A0, A1, A2, A3)]
    tab_cat = jnp.concatenate(tabs, axis=1)
    idx_all = jnp.concatenate(
        [x.reshape(S_STORY, SENT), query], axis=0).reshape(-1)
    a3t = tabs[3].T

    e0, e1, e2, e3 = _sc_embed(idx_all, tab_cat)
    u0 = e0[S_STORY:]
    est = [e[:S_STORY].reshape(BS, STORY, DPAD) for e in (e0, e1, e2, e3)]

    out = pl.pallas_call(
        _tc_dense,
        grid=(BS // BB,),
        in_specs=[
            pl.BlockSpec((BB, DPAD), lambda g: (g, 0)),
            *[pl.BlockSpec((BB, STORY, DPAD), lambda g: (g, 0, 0))
              for _ in range(NT)],
            pl.BlockSpec((DPAD, VOCAB), lambda g: (0, 0)),
        ],
        out_specs=pl.BlockSpec((BB, VOCAB), lambda g: (g, 0)),
        out_shape=jax.ShapeDtypeStruct((BS, VOCAB), jnp.float32),
    )(u0, *est, a3t)
    return out


# speedup vs baseline: 13.5616x; 1.2717x over previous
import functools

import jax
import jax.numpy as jnp
from jax import lax
from jax.experimental import pallas as pl
from jax.experimental.pallas import tpu as pltpu
from jax.experimental.pallas import tpu_sc as plsc

VOCAB = 1000
EMBD = 150
HOPS = 3
BS = 1024
STORY = 50
SENT = 20
QLEN = 20

DPAD = 160
NT = HOPS + 1
DCAT = NT * DPAD
NCH = DPAD // 16

NC = 2
NS = 16
NW = NC * NS

B_PER_W = BS // NW
B_SEG = 2
NIDX = B_SEG * SENT
ST_PER_B = STORY // B_SEG
N_ST = B_PER_W * ST_PER_B
N_Q = B_PER_W // B_SEG

_mesh = plsc.VectorSubcoreMesh(core_axis_name="c", subcore_axis_name="s")


def _lane_shuffle(v, shift):
    perm = (lax.iota(jnp.int32, 16) + shift) % 16
    dnums = lax.GatherDimensionNumbers(
        offset_dims=(), collapsed_slice_dims=(0,), start_index_map=(0,))
    return lax.gather(v, perm[:, None], dnums, slice_sizes=(1,),
                      mode=lax.GatherScatterMode.PROMISE_IN_BOUNDS)


def _lane_sum(v):
    for sh in (8, 4, 2, 1):
        v = v + _lane_shuffle(v, sh)
    return v


def _lane_max(v):
    for sh in (8, 4, 2, 1):
        v = jnp.maximum(v, _lane_shuffle(v, sh))
    return v


@functools.partial(
    pl.kernel,
    out_type=jax.ShapeDtypeStruct((BS, DPAD), jnp.float32),
    mesh=_mesh,
    scratch_types=[
        pltpu.VMEM((B_PER_W * QLEN,), jnp.int32),
        pltpu.VMEM((STORY * SENT,), jnp.int32),
        pltpu.VMEM((STORY * SENT,), jnp.int32),
        pltpu.VMEM((NIDX, DCAT), jnp.float32),
        pltpu.VMEM((NIDX, DCAT), jnp.float32),
        pltpu.VMEM((STORY, DCAT), jnp.float32),
        pltpu.VMEM((B_PER_W, DPAD), jnp.float32),
        pltpu.VMEM((DPAD,), jnp.float32),
        pltpu.VMEM((64,), jnp.float32),
        pltpu.SemaphoreType.DMA,
        pltpu.SemaphoreType.DMA,
        pltpu.SemaphoreType.DMA,
        pltpu.SemaphoreType.DMA,
    ],
)
def _sc_forward(xf_hbm, qf_hbm, tab_hbm, u3_hbm,
                qidx_v, bidx0, bidx1, rows0, rows1, e_v, ustage, u_v, p_v,
                sem_r0, sem_r1, sem_i0, sem_i1):
    wid = lax.axis_index("s") * NC + lax.axis_index("c")
    rows = (rows0, rows1)
    sems = (sem_r0, sem_r1)
    bidx = (bidx0, bidx1)
    isems = (sem_i0, sem_i1)
    b0 = wid * B_PER_W

    def gather(idx_ref, rbuf, sem):
        pltpu.async_copy(tab_hbm.at[idx_ref], rbuf, sem)

    def wait_rows(rbuf, sem):
        pltpu.make_async_copy(tab_hbm.at[qidx_v.at[pl.ds(0, NIDX)]],
                              rbuf, sem).wait()

    def wait_idx(which):
        pltpu.make_async_copy(xf_hbm.at[pl.ds(0, STORY * SENT)],
                              bidx[which], isems[which]).wait()

    pltpu.sync_copy(qf_hbm.at[pl.ds(b0 * QLEN, B_PER_W * QLEN)], qidx_v)
    gather(qidx_v.at[pl.ds(0, NIDX)], rows0, sem_r0)

    def qstep(q, carry):
        par = q % 2
        for pp in range(2):
            @pl.when(par == pp)
            def _(pp=pp):
                wait_rows(rows[pp], sems[pp])

                @pl.when(q + 1 < N_Q)
                def _():
                    gather(qidx_v.at[pl.ds((q + 1) * NIDX, NIDX)],
                           rows[1 - pp], sems[1 - pp])
                rbuf = rows[pp]
                for s in range(B_SEG):
                    def chunk(d, _, s=s, rbuf=rbuf):
                        col = d * 16
                        a = rbuf[s * SENT, pl.ds(col, 16)]
                        b = rbuf[s * SENT + 1, pl.ds(col, 16)]
                        for t in range(2, SENT, 2):
                            a = a + rbuf[s * SENT + t, pl.ds(col, 16)]
                            b = b + rbuf[s * SENT + t + 1, pl.ds(col, 16)]
                        ustage[q * B_SEG + s, pl.ds(col, 16)] = a + b
                        return 0
                    lax.fori_loop(0, NCH, chunk, 0)
        return carry

    lax.fori_loop(0, N_Q, qstep, 0)

    pltpu.sync_copy(xf_hbm.at[pl.ds(b0 * STORY * SENT, STORY * SENT)], bidx0)
    pltpu.async_copy(xf_hbm.at[pl.ds((b0 + 1) * STORY * SENT, STORY * SENT)],
                     bidx1, sem_i1)
    gather(bidx0.at[pl.ds(0, NIDX)], rows0, sem_r0)

    def hops(bi):
        for d in range(NCH):
            u_v[pl.ds(d * 16, 16)] = ustage[bi, pl.ds(d * 16, 16)]
        for k in range(HOPS):
            mcol = k * DPAD
            ccol = (k + 1) * DPAD

            for g in range(4):
                lo = g * 16
                hi = min(STORY, lo + 16)
                if g < 3:
                    init = jnp.zeros((16,), jnp.float32)
                else:
                    init = jnp.where(lax.iota(jnp.int32, 16) < STORY - lo,
                                     0.0, -1e30)

                def dots(s, pacc, lo=lo):
                    acc = e_v[s, pl.ds(mcol, 16)] * u_v[pl.ds(0, 16)]
                    for d in range(1, NCH):
                        acc = acc + (e_v[s, pl.ds(mcol + d * 16, 16)]
                                     * u_v[pl.ds(d * 16, 16)])
                    ps = _lane_sum(acc)
                    return jnp.where(
                        lax.iota(jnp.int32, 16) == s - lo, ps, pacc)
                p_v[pl.ds(lo, 16)] = lax.fori_loop(lo, hi, dots, init)

            c0 = p_v[pl.ds(0, 16)]
            c1 = p_v[pl.ds(16, 16)]
            c2 = p_v[pl.ds(32, 16)]
            c3 = p_v[pl.ds(48, 16)]
            mx = _lane_max(jnp.maximum(jnp.maximum(c0, c1),
                                       jnp.maximum(c2, c3)))
            e0 = jnp.exp(c0 - mx)
            e1 = jnp.exp(c1 - mx)
            e2 = jnp.exp(c2 - mx)
            e3 = jnp.exp(c3 - mx)
            r = 1.0 / _lane_sum(e0 + e1 + e2 + e3)
            p_v[pl.ds(0, 16)] = e0 * r
            p_v[pl.ds(16, 16)] = e1 * r
            p_v[pl.ds(32, 16)] = e2 * r
            p_v[pl.ds(48, 16)] = e3 * r

            def wsum_chunk(d, carry):
                pg = [p_v[pl.ds(16 * g, 16)] for g in range(4)]
                acc0 = jnp.zeros((16,), jnp.float32)
                acc1 = jnp.zeros((16,), jnp.float32)
                for s in range(0, STORY, 2):
                    acc0 = acc0 + (pg[s // 16][s % 16]
                                   * e_v[s, pl.ds(ccol + d * 16, 16)])
                    s1 = s + 1
                    acc1 = acc1 + (pg[s1 // 16][s1 % 16]
                                   * e_v[s1, pl.ds(ccol + d * 16, 16)])
                u_v[pl.ds(d * 16, 16)] = (u_v[pl.ds(d * 16, 16)]
                                          + acc0 + acc1)
                return carry
            lax.fori_loop(0, NCH, wsum_chunk, 0)
        pltpu.sync_copy(u_v, u3_hbm.at[b0 + bi])

    def sstep(st, carry):
        bi = st // ST_PER_B
        slot = st - bi * ST_PER_B
        par = st % 2
        for pp in range(2):
            @pl.when(par == pp)
            def _(pp=pp):
                wait_rows(rows[pp], sems[pp])

                @pl.when(slot + 1 < ST_PER_B)
                def _(pp=pp):
                    for bp in range(2):
                        @pl.when(bi % 2 == bp)
                        def _(pp=pp, bp=bp):
                            gather(bidx[bp].at[pl.ds((slot + 1) * NIDX, NIDX)],
                                   rows[1 - pp], sems[1 - pp])

                @pl.when(jnp.logical_and(slot + 1 == ST_PER_B,
                                         st + 1 < N_ST))
                def _(pp=pp):
                    for bp in range(2):
                        @pl.when((bi + 1) % 2 == bp)
                        def _(pp=pp, bp=bp):
                            wait_idx(bp)
                            gather(bidx[bp].at[pl.ds(0, NIDX)],
                                   rows[1 - pp], sems[1 - pp])

                rbuf = rows[pp]
                for s in range(B_SEG):
                    def chunk(d, _, s=s, rbuf=rbuf):
                        col = d * 16
                        a = rbuf[s * SENT, pl.ds(col, 16)]
                        b = rbuf[s * SENT + 1, pl.ds(col, 16)]
                        for t in range(2, SENT, 2):
                            a = a + rbuf[s * SENT + t, pl.ds(col, 16)]
                            b = b + rbuf[s * SENT + t + 1, pl.ds(col, 16)]
                        e_v[slot * B_SEG + s, pl.ds(col, 16)] = a + b
                        return 0
                    lax.fori_loop(0, NCH * NT, chunk, 0)

        @pl.when(slot + 1 == ST_PER_B)
        def _():
            hops(bi)

            @pl.when(bi + 2 < B_PER_W)
            def _():
                for bp in range(2):
                    @pl.when(bi % 2 == bp)
                    def _(bp=bp):
                        pltpu.async_copy(
                            xf_hbm.at[pl.ds((b0 + bi + 2) * STORY * SENT,
                                            STORY * SENT)],
                            bidx[bp], isems[bp])
        return carry

    lax.fori_loop(0, N_ST, sstep, 0)


BBT = 256


def _tc_final(u3_ref, a3t_ref, out_ref):
    logits = jnp.dot(u3_ref[...], a3t_ref[...],
                     preferred_element_type=jnp.float32)
    logits = logits - jnp.max(logits, axis=1, keepdims=True)
    z = jnp.exp(logits)
    out_ref[...] = z / jnp.sum(z, axis=1, keepdims=True)


def kernel(x, query, A0, A1, A2, A3):
    tabs = [jnp.pad(t, ((0, 0), (0, DPAD - EMBD))) for t in (A0, A1, A2, A3)]
    tab_cat = jnp.concatenate(tabs, axis=1)
    a3t = tabs[3].T
    xf = x.reshape(-1)
    qf = query.reshape(-1)

    u3 = _sc_forward(xf, qf, tab_cat)

    out = pl.pallas_call(
        _tc_final,
        grid=(BS // BBT,),
        in_specs=[
            pl.BlockSpec((BBT, DPAD), lambda g: (g, 0)),
            pl.BlockSpec((DPAD, VOCAB), lambda g: (0, 0)),
        ],
        out_specs=pl.BlockSpec((BBT, VOCAB), lambda g: (g, 0)),
        out_shape=jax.ShapeDtypeStruct((BS, VOCAB), jnp.float32),
    )(u3, a3t)
    return out
